# baseline (speedup 1.0000x reference)
import functools

import jax
import jax.numpy as jnp
from jax import lax
from jax.experimental import pallas as pl
from jax.experimental.pallas import tpu as pltpu
from jax.experimental.pallas import tpu_sc as plsc

_VOCAB = 1000000
_DIM = 128
_MAXPOS = 2048
_B = 4
_S = 2048
_EPS = 1e-12

_NC = 2
_NS = 16
_NW = _NC * _NS
_N = _B * _S
_TPW = _N // _NW
_WPB = _S // _TPW
_IDR = _TPW // 128


def _pos_table():
    exponents = -jnp.arange(0, _DIM, 2, dtype=jnp.float32) * jnp.log(10000.0)
    deno = jnp.exp(-exponents / _DIM)[None, :]
    pos = jnp.arange(0, _MAXPOS, dtype=jnp.float32)[:, None]
    args = pos * deno
    emb = jnp.zeros((_MAXPOS, _DIM), dtype=jnp.float32)
    emb = emb.at[:, 0::2].set(jnp.sin(args))
    emb = emb.at[:, 1::2].set(jnp.cos(args))
    return emb

_V8 = _DIM // 16


@functools.partial(
    pl.kernel,
    out_type=jax.ShapeDtypeStruct((_N, _DIM), jnp.float32),
    mesh=plsc.VectorSubcoreMesh(core_axis_name="c", subcore_axis_name="s"),
    scratch_types=[
        pltpu.VMEM((_IDR, 128), jnp.int32),
        pltpu.VMEM((_IDR, 128), jnp.int32),
        pltpu.VMEM((_TPW, _DIM), jnp.float32),
        pltpu.VMEM((_TPW, _DIM), jnp.float32),
        pltpu.VMEM((_TPW, _DIM), jnp.float32),
        pltpu.VMEM((_DIM,), jnp.float32),
        pltpu.VMEM((_DIM,), jnp.float32),
        pltpu.SemaphoreType.DMA,
    ],
)
def _embed_ln(ids, tt, word, ttab, pos, gamma, beta, out,
              idx_v, ttv, wbuf, tbuf, pbuf, gbuf, bbuf, sem):
    wid = lax.axis_index("s") * _NC + lax.axis_index("c")
    base = wid * _TPW
    poff = lax.rem(wid, _WPB) * _TPW
    rbase = wid * _IDR

    pltpu.sync_copy(ids.at[pl.ds(rbase, _IDR)], idx_v)
    pltpu.sync_copy(tt.at[pl.ds(rbase, _IDR)], ttv)
    copies = []
    for j in range(_IDR):
        copies.append(pltpu.async_copy(
            word.at[idx_v.at[j]], wbuf.at[pl.ds(j * 128, 128)], sem))
        copies.append(pltpu.async_copy(
            ttab.at[ttv.at[j]], tbuf.at[pl.ds(j * 128, 128)], sem))
    pltpu.sync_copy(pos.at[pl.ds(poff, _TPW)], pbuf)
    pltpu.sync_copy(gamma, gbuf)
    pltpu.sync_copy(beta, bbuf)
    for c in copies:
        c.wait()

    inv_d = jnp.float32(1.0 / _DIM)

    def lanesum(x):
        for k in (8, 4, 2, 1):
            idx = jnp.bitwise_xor(lax.iota(jnp.int32, 16), jnp.int32(k))
            x = x + x.at[idx].get(mode="promise_in_bounds")
        return x

    def token(t, carry):
        x = []
        for v in range(_V8):
            sl = pl.ds(v * 16, 16)
            x.append(wbuf[t, sl] + pbuf[t, sl] + tbuf[t, sl])
        s = x[0]
        q = x[0] * x[0]
        for v in range(1, _V8):
            s = s + x[v]
            q = q + x[v] * x[v]
        meanv = lanesum(s) * inv_d
        msqv = lanesum(q) * inv_d
        vvec = msqv - meanv * meanv + jnp.float32(_EPS)
        bits = lax.bitcast_convert_type(vvec, jnp.int32)
        guess = jnp.int32(0x5F3759DF) - lax.shift_right_logical(bits, 1)
        y = lax.bitcast_convert_type(guess, jnp.float32)
        for _ in range(3):
            y = y * (jnp.float32(1.5) - jnp.float32(0.5) * vvec * y * y)
        for v in range(_V8):
            sl = pl.ds(v * 16, 16)
            wbuf[t, sl] = (x[v] - meanv) * y * gbuf[sl] + bbuf[sl]
        return carry

    lax.fori_loop(0, _TPW, token, 0)
    pltpu.sync_copy(wbuf, out.at[pl.ds(base, _TPW)])


def kernel(input_ids, token_type_ids, word_table, type_table, gamma, beta):
    ids = input_ids.astype(jnp.int32).reshape(_N // 128, 128)
    tt = token_type_ids.astype(jnp.int32).reshape(_N // 128, 128)
    pos = _pos_table()
    out = _embed_ln(ids, tt, word_table, type_table, pos, gamma, beta)
    return out.reshape(_B, _S, _DIM)


# speedup vs baseline: 5.0939x; 5.0939x over previous
import functools

import jax
import jax.numpy as jnp
from jax import lax
from jax.experimental import pallas as pl
from jax.experimental.pallas import tpu as pltpu
from jax.experimental.pallas import tpu_sc as plsc

_VOCAB = 1000000
_DIM = 128
_MAXPOS = 2048
_B = 4
_S = 2048
_EPS = 1e-12

_NC = 2
_NS = 16
_NW = _NC * _NS
_N = _B * _S
_TPW = _N // _NW
_IDR = _TPW // 128

_ROWS_PER_BLK = 512
_GRID = _N // _ROWS_PER_BLK
_POS_BLKS = _S // _ROWS_PER_BLK


def _pos_table():
    exponents = -jnp.arange(0, _DIM, 2, dtype=jnp.float32) * jnp.log(10000.0)
    deno = jnp.exp(-exponents / _DIM)[None, :]
    pos = jnp.arange(0, _MAXPOS, dtype=jnp.float32)[:, None]
    args = pos * deno
    emb = jnp.zeros((_MAXPOS, _DIM), dtype=jnp.float32)
    emb = emb.at[:, 0::2].set(jnp.sin(args))
    emb = emb.at[:, 1::2].set(jnp.cos(args))
    return emb


@functools.partial(
    pl.kernel,
    out_type=jax.ShapeDtypeStruct((_N, _DIM), jnp.float32),
    mesh=plsc.VectorSubcoreMesh(core_axis_name="c", subcore_axis_name="s"),
    scratch_types=[
        pltpu.VMEM((_IDR, 128), jnp.int32),
        pltpu.VMEM((_TPW, _DIM), jnp.float32),
        pltpu.SemaphoreType.DMA,
    ],
)
def _gather_rows(ids, word, out, idx_v, wbuf, sem):
    wid = lax.axis_index("s") * _NC + lax.axis_index("c")
    pltpu.sync_copy(ids.at[pl.ds(wid * _IDR, _IDR)], idx_v)
    copies = [
        pltpu.async_copy(word.at[idx_v.at[j]], wbuf.at[pl.ds(j * 128, 128)], sem)
        for j in range(_IDR)
    ]
    for c in copies:
        c.wait()
    pltpu.sync_copy(wbuf, out.at[pl.ds(wid * _TPW, _TPW)])


def _ln_body(g_ref, pos_ref, ttf_ref, ttab_ref, gam_ref, bet_ref, o_ref):
    r0 = ttab_ref[0:1, :]
    r1 = ttab_ref[1:2, :]
    x = g_ref[...] + pos_ref[...] + (r0 + ttf_ref[...] * (r1 - r0))
    mean = jnp.mean(x, axis=-1, keepdims=True)
    xc = x - mean
    var = jnp.mean(xc * xc, axis=-1, keepdims=True)
    o_ref[...] = xc * lax.rsqrt(var + _EPS) * gam_ref[...] + bet_ref[...]


_ln_call = pl.pallas_call(
    _ln_body,
    grid=(_GRID,),
    in_specs=[
        pl.BlockSpec((_ROWS_PER_BLK, _DIM), lambda j: (j, 0)),
        pl.BlockSpec((_ROWS_PER_BLK, _DIM), lambda j: (j % _POS_BLKS, 0)),
        pl.BlockSpec((_ROWS_PER_BLK, 1), lambda j: (j, 0)),
        pl.BlockSpec((2, _DIM), lambda j: (0, 0)),
        pl.BlockSpec((1, _DIM), lambda j: (0, 0)),
        pl.BlockSpec((1, _DIM), lambda j: (0, 0)),
    ],
    out_specs=pl.BlockSpec((_ROWS_PER_BLK, _DIM), lambda j: (j, 0)),
    out_shape=jax.ShapeDtypeStruct((_N, _DIM), jnp.float32),
)


def kernel(input_ids, token_type_ids, word_table, type_table, gamma, beta):
    ids = input_ids.astype(jnp.int32).reshape(_N // 128, 128)
    g = _gather_rows(ids, word_table)
    ttf = token_type_ids.reshape(_N, 1).astype(jnp.float32)
    out = _ln_call(g, _pos_table(), ttf, type_table,
                   gamma.reshape(1, _DIM), beta.reshape(1, _DIM))
    return out.reshape(_B, _S, _DIM)


# speedup vs baseline: 6.0725x; 1.1921x over previous
import functools

import jax
import jax.numpy as jnp
from jax import lax
from jax.experimental import pallas as pl
from jax.experimental.pallas import tpu as pltpu
from jax.experimental.pallas import tpu_sc as plsc

_VOCAB = 1000000
_DIM = 128
_MAXPOS = 2048
_B = 4
_S = 2048
_EPS = 1e-12

_NC = 2
_NS = 16
_NW = _NC * _NS
_N = _B * _S
_TPW = _N // _NW
_IDR = _TPW // 128

_ROWS_PER_BLK = 2048
_GRID = _N // _ROWS_PER_BLK


def _pos_table():
    exponents = -jnp.arange(0, _DIM, 2, dtype=jnp.float32) * jnp.log(10000.0)
    deno = jnp.exp(-exponents / _DIM)[None, :]
    pos = jnp.arange(0, _MAXPOS, dtype=jnp.float32)[:, None]
    args = pos * deno
    emb = jnp.zeros((_MAXPOS, _DIM), dtype=jnp.float32)
    emb = emb.at[:, 0::2].set(jnp.sin(args))
    emb = emb.at[:, 1::2].set(jnp.cos(args))
    return emb


@functools.partial(
    pl.kernel,
    out_type=jax.ShapeDtypeStruct((_N, _DIM), jnp.float32),
    mesh=plsc.VectorSubcoreMesh(core_axis_name="c", subcore_axis_name="s"),
    scratch_types=[
        pltpu.VMEM((_IDR, 128), jnp.int32),
        pltpu.VMEM((_TPW, _DIM), jnp.float32),
        pltpu.SemaphoreType.DMA,
    ],
)
def _gather_rows(ids, word, out, idx_v, wbuf, sem):
    wid = lax.axis_index("s") * _NC + lax.axis_index("c")
    pltpu.sync_copy(ids.at[pl.ds(wid * _IDR, _IDR)], idx_v)
    copies = [
        pltpu.async_copy(word.at[idx_v.at[j]], wbuf.at[pl.ds(j * 128, 128)], sem)
        for j in range(_IDR)
    ]
    for c in copies:
        c.wait()
    pltpu.sync_copy(wbuf, out.at[pl.ds(wid * _TPW, _TPW)])


def _ln_body(g_ref, pos_ref, ttf_ref, ttab_ref, gam_ref, bet_ref, o_ref):
    r0 = ttab_ref[0:1, :]
    r1 = ttab_ref[1:2, :]
    x = g_ref[...] + pos_ref[...] + (r0 + ttf_ref[...] * (r1 - r0))
    mean = jnp.mean(x, axis=-1, keepdims=True)
    xc = x - mean
    var = jnp.mean(xc * xc, axis=-1, keepdims=True)
    o_ref[...] = xc * lax.rsqrt(var + _EPS) * gam_ref[...] + bet_ref[...]


_ln_call = pl.pallas_call(
    _ln_body,
    grid=(_GRID,),
    in_specs=[
        pl.BlockSpec((_ROWS_PER_BLK, _DIM), lambda j: (j, 0)),
        pl.BlockSpec((_S, _DIM), lambda j: (0, 0)),
        pl.BlockSpec((_ROWS_PER_BLK, 1), lambda j: (j, 0)),
        pl.BlockSpec((2, _DIM), lambda j: (0, 0)),
        pl.BlockSpec((1, _DIM), lambda j: (0, 0)),
        pl.BlockSpec((1, _DIM), lambda j: (0, 0)),
    ],
    out_specs=pl.BlockSpec((_ROWS_PER_BLK, _DIM), lambda j: (j, 0)),
    out_shape=jax.ShapeDtypeStruct((_N, _DIM), jnp.float32),
)


def kernel(input_ids, token_type_ids, word_table, type_table, gamma, beta):
    ids = input_ids.astype(jnp.int32).reshape(_N // 128, 128)
    g = _gather_rows(ids, word_table)
    ttf = token_type_ids.reshape(_N, 1).astype(jnp.float32)
    out = _ln_call(g, _pos_table(), ttf, type_table,
                   gamma.reshape(1, _DIM), beta.reshape(1, _DIM))
    return out.reshape(_B, _S, _DIM)
